# baseline (speedup 1.0000x reference)
import jax
import jax.numpy as jnp
from jax import lax
from jax.experimental import pallas as pl
from jax.experimental.pallas import tpu as pltpu
from jax.experimental.pallas import tpu_sc as plsc


NT = 1024
GRID = 4096 // NT


def _sc_route(sl_hbm, il_hbm, ss_hbm, is_hbm, ol_hbm, os_hbm,
              idxl_v, rowsl_v, idxs_v, rowss_v, sem):
    w = lax.axis_index("s") * 2 + lax.axis_index("c")
    base = w * 32
    pltpu.sync_copy(il_hbm.at[pl.ds(base, 32)], idxl_v)
    pltpu.async_copy(sl_hbm.at[idxl_v], rowsl_v, sem).wait()
    pltpu.sync_copy(rowsl_v, ol_hbm.at[pl.ds(base, 32)])
    pltpu.sync_copy(is_hbm.at[pl.ds(base, 32)], idxs_v)
    pltpu.async_copy(ss_hbm.at[idxs_v], rowss_v, sem).wait()
    pltpu.sync_copy(rowss_v, os_hbm.at[pl.ds(base, 32)])


_sc_route_call = pl.kernel(
    _sc_route,
    mesh=plsc.VectorSubcoreMesh(core_axis_name="c", subcore_axis_name="s"),
    out_type=[
        jax.ShapeDtypeStruct((1024, 128), jnp.float32),
        jax.ShapeDtypeStruct((1024, 128), jnp.float32),
    ],
    scratch_types=[
        pltpu.VMEM((32,), jnp.int32),
        pltpu.VMEM((32, 128), jnp.float32),
        pltpu.VMEM((32,), jnp.int32),
        pltpu.VMEM((32, 128), jnp.float32),
        pltpu.SemaphoreType.DMA,
    ],
)


def _encode(z):
    v = jax.lax.bitcast_convert_type(z, jnp.int32)
    s = (v >> 16) & 0x8000
    a = (v & 0x7FFFFFFF) + 0x1000
    u = jnp.maximum(a - 0x38000000, 0)
    return s | (u >> 13)


def _body(yscl_ref, yscs_ref, bl_ref, bs_ref, out_ref):
    ysc_l = yscl_ref[...].astype(jnp.bfloat16)
    ysc_s = yscs_ref[...].astype(jnp.bfloat16)
    dn = (((1,), (0,)), ((), ()))
    zl = jax.lax.dot_general(ysc_l, bl_ref[...], dn,
                             preferred_element_type=jnp.float32) * 2.0
    zs = jax.lax.dot_general(ysc_s, bs_ref[...], dn,
                             preferred_element_type=jnp.float32) * 2.0

    ob = out_ref.bitcast(jnp.int32)
    ob[0:64, :] = _encode(zl[0:64]) | (_encode(zl[64:128]) << 16)
    ob[64:128, :] = _encode(zs[0:64]) | (_encode(zs[64:128]) << 16)


@jax.jit
def kernel(y_large, y_small, wids_large, wids_small, lora_B_large, lora_B_small):
    perm = jnp.concatenate([jnp.arange(0, 128, 2, dtype=jnp.int32),
                            jnp.arange(1, 128, 2, dtype=jnp.int32)])
    ylp = y_large.reshape(128, 64)[perm].astype(jnp.float32)
    ysp = y_small.reshape(128, 16)[perm].astype(jnp.float32)
    wlp = wids_large[perm]
    wsp = wids_small[perm]

    tok = jnp.repeat(jnp.arange(128, dtype=jnp.int32), 8)
    sub = jnp.tile(jnp.arange(8, dtype=jnp.int32), 128)
    wl8 = jnp.repeat(wlp, 8)
    il = jnp.where((wl8 >> 1) == sub, (wl8 & 1) * 128 + tok, 256)
    z64 = jnp.zeros((128, 64), jnp.float32)
    sl = jnp.concatenate([
        jnp.concatenate([ylp, z64], 1),
        jnp.concatenate([z64, ylp], 1),
        jnp.zeros((8, 128), jnp.float32),
    ])

    ws8 = jnp.repeat(wsp, 8)
    isv = jnp.where((ws8 >> 3) == sub, (ws8 & 7) * 128 + tok, 1024)
    z16 = jnp.zeros((128, 16), jnp.float32)
    banks = [jnp.concatenate([z16] * p + [ysp] + [z16] * (7 - p), 1)
             for p in range(8)]
    ss = jnp.concatenate(banks + [jnp.zeros((8, 128), jnp.float32)])

    yscl, yscs = _sc_route_call(sl, il, ss, isv)
    yscl = yscl.reshape(128, 1024)
    yscs = yscs.reshape(128, 1024)

    bl = lora_B_large.reshape(16 * 64, 4096).astype(jnp.bfloat16)
    bs = lora_B_small.reshape(64 * 16, 4096).astype(jnp.bfloat16)

    out = pl.pallas_call(
        _body,
        grid=(GRID,),
        in_specs=[
            pl.BlockSpec((128, 1024), lambda n: (0, 0)),
            pl.BlockSpec((128, 1024), lambda n: (0, 0)),
            pl.BlockSpec((1024, NT), lambda n: (0, n)),
            pl.BlockSpec((1024, NT), lambda n: (0, n)),
        ],
        out_specs=pl.BlockSpec((256, NT), lambda n: (0, n)),
        out_shape=jax.ShapeDtypeStruct((256, 4096), jnp.bfloat16),
    )(yscl, yscs, bl, bs)
    z = jax.lax.bitcast_convert_type(out, jnp.float16)
    return z.reshape(256, 1, 4096)


# speedup vs baseline: 2.7192x; 2.7192x over previous
import jax
import jax.numpy as jnp
from jax.experimental import pallas as pl
from jax.experimental.pallas import tpu as pltpu


NT = 1024
GRID = 4096 // NT


def _encode(z):
    v = jax.lax.bitcast_convert_type(z, jnp.int32)
    s = (v >> 16) & 0x8000
    a = (v & 0x7FFFFFFF) + 0x1000
    u = jnp.maximum(a - 0x38000000, 0)
    return s | (u >> 13)


def _body(yl_ref, ys_ref, wl_ref, ws_ref, bl_ref, bs_ref, out_ref,
          yscl_scr, yscs_scr):
    @pl.when(pl.program_id(0) == 0)
    def _init():
        iota = jax.lax.broadcasted_iota(jnp.int32, (128, 1024), 1)
        zero = jnp.bfloat16(0)
        yl = yl_ref[...].astype(jnp.bfloat16)
        t_l = jnp.concatenate([yl] * 16, axis=1)
        yscl_scr[...] = jnp.where((iota >> 6) == wl_ref[...], t_l, zero)
        ys = ys_ref[...].astype(jnp.bfloat16)
        t_s = jnp.concatenate([ys] * 64, axis=1)
        yscs_scr[...] = jnp.where((iota >> 4) == ws_ref[...], t_s, zero)

    dn = (((1,), (0,)), ((), ()))
    zl = jax.lax.dot_general(yscl_scr[...], bl_ref[...], dn,
                             preferred_element_type=jnp.float32) * 2.0
    zs = jax.lax.dot_general(yscs_scr[...], bs_ref[...], dn,
                             preferred_element_type=jnp.float32) * 2.0

    ob = out_ref.bitcast(jnp.int32)
    ob[0:64, :] = _encode(zl[0:64]) | (_encode(zl[64:128]) << 16)
    ob[64:128, :] = _encode(zs[0:64]) | (_encode(zs[64:128]) << 16)


@jax.jit
def kernel(y_large, y_small, wids_large, wids_small, lora_B_large, lora_B_small):
    perm = jnp.concatenate([jnp.arange(0, 128, 2, dtype=jnp.int32),
                            jnp.arange(1, 128, 2, dtype=jnp.int32)])
    ylp = y_large.reshape(128, 64)[perm].astype(jnp.float32)
    ysp = y_small.reshape(128, 16)[perm].astype(jnp.float32)
    wl = wids_large[perm].reshape(128, 1)
    ws = wids_small[perm].reshape(128, 1)
    bl = lora_B_large.reshape(16 * 64, 4096).astype(jnp.bfloat16)
    bs = lora_B_small.reshape(64 * 16, 4096).astype(jnp.bfloat16)

    out = pl.pallas_call(
        _body,
        grid=(GRID,),
        in_specs=[
            pl.BlockSpec((128, 64), lambda n: (0, 0)),
            pl.BlockSpec((128, 16), lambda n: (0, 0)),
            pl.BlockSpec((128, 1), lambda n: (0, 0)),
            pl.BlockSpec((128, 1), lambda n: (0, 0)),
            pl.BlockSpec((1024, NT), lambda n: (0, n)),
            pl.BlockSpec((1024, NT), lambda n: (0, n)),
        ],
        out_specs=pl.BlockSpec((256, NT), lambda n: (0, n)),
        out_shape=jax.ShapeDtypeStruct((256, 4096), jnp.bfloat16),
        scratch_shapes=[
            pltpu.VMEM((128, 1024), jnp.bfloat16),
            pltpu.VMEM((128, 1024), jnp.bfloat16),
        ],
    )(ylp, ysp, wl, ws, bl, bs)
    z = jax.lax.bitcast_convert_type(out, jnp.float16)
    return z.reshape(256, 1, 4096)
